# Initial kernel scaffold; baseline (speedup 1.0000x reference)
import jax
import jax.numpy as jnp
from jax.experimental import pallas as pl


def kernel(e1s, e2s, e3s, r1s, r2s, node_embedding, node_type, node_neighbors, rel_neighbors, node_weight, rel_weight, type_weight, rel_eye):
    raise NotImplementedError("write your pallas kernel here")


# speedup vs baseline: 7.7200x; 7.7200x over previous
import functools

import jax
import jax.numpy as jnp
from jax import lax
from jax.experimental import pallas as pl
from jax.experimental.pallas import tpu as pltpu
from jax.experimental.pallas import tpu_sc as plsc

_NW = 32
_L = 16
_NMAX_PAD = 99999


def _butterfly(x, op):
  lane = lax.iota(jnp.int32, _L)
  for sh in (8, 4, 2, 1):
    x = op(x, x.at[lane ^ sh].get(mode="promise_in_bounds"))
  return x


def _allsum(x):
  return _butterfly(x, lambda a, b: a + b)


def _allmax(x):
  return _butterfly(x, jnp.maximum)


def _sc_body(e2s_h, e3s_h, emb_h, ntype_h, nbrs_h, nrels_h, nodew_h,
             edge_h, typew_h, out_h,
             e2i, e3i, nbi, nri, e2e, e3e, nbe, nwv, ntv, ntc,
             edgv, typv, lossv, sem):
  QPW = e2i.shape[0]
  NN = nbi.shape[1]
  D = e2e.shape[1]
  ND = D // _L

  wid = lax.axis_index("s") * 2 + lax.axis_index("c")
  base = wid * QPW

  pltpu.sync_copy(e2s_h.at[pl.ds(base, QPW)], e2i)
  pltpu.sync_copy(e3s_h.at[pl.ds(base, QPW)], e3i)

  for c in range(QPW // 128):
    sl = pl.ds(c * 128, 128)
    h1 = pltpu.async_copy(nbrs_h.at[e2i.at[sl]], nbi.at[sl], sem)
    h2 = pltpu.async_copy(nrels_h.at[e2i.at[sl]], nri.at[sl], sem)
    h3 = pltpu.async_copy(emb_h.at[e2i.at[sl]], e2e.at[sl], sem)
    h4 = pltpu.async_copy(emb_h.at[e3i.at[sl]], e3e.at[sl], sem)
    h1.wait()
    h2.wait()
    h3.wait()
    h4.wait()

  def per_query(q, carry):
    g1 = pltpu.async_copy(emb_h.at[nbi.at[q]], nbe, sem)
    g2 = pltpu.async_copy(nodew_h.at[nbi.at[q]], nwv, sem)
    nb_lo = nbi[q, pl.ds(0, _L)]
    nb_hi = nbi[q, pl.ds(_L, _L)]
    ntc[pl.ds(0, _L)] = jnp.minimum(nb_lo, _NMAX_PAD)
    ntc[pl.ds(_L, _L)] = jnp.minimum(nb_hi, _NMAX_PAD)
    g3 = pltpu.async_copy(ntype_h.at[ntc], ntv, sem)
    g4 = pltpu.async_copy(edge_h.at[nri.at[q]], edgv, sem)

    e2r = [e2e[q, pl.ds(k * _L, _L)] for k in range(ND)]
    e3r = [e3e[q, pl.ds(k * _L, _L)] for k in range(ND)]
    acc = (e2r[0] - e3r[0]) * (e2r[0] - e3r[0])
    for k in range(1, ND):
      d = e2r[k] - e3r[k]
      acc = acc + d * d
    d3v = _allsum(acc)

    g3.wait()
    g5 = pltpu.async_copy(typew_h.at[ntv], typv, sem)
    g2.wait()
    g4.wait()
    g5.wait()
    logit_lo = nwv[pl.ds(0, _L)] + edgv[pl.ds(0, _L)] + typv[pl.ds(0, _L)]
    logit_hi = nwv[pl.ds(_L, _L)] + edgv[pl.ds(_L, _L)] + typv[pl.ds(_L, _L)]

    m = _allmax(jnp.maximum(logit_lo, logit_hi))
    el = jnp.exp(logit_lo - m)
    eh = jnp.exp(logit_hi - m)
    s = _allsum(el + eh)
    wlo = el / s
    whi = eh / s

    g1.wait()

    coef = jnp.zeros((_L,), jnp.float32)
    accrow = [jnp.zeros((_L,), jnp.float32) for _ in range(ND)]
    for n in range(NN):
      crow = [nbe[n, pl.ds(k * _L, _L)] for k in range(ND)]
      dd = (crow[0] - e3r[0]) * (crow[0] - e3r[0])
      for k in range(1, ND):
        d = crow[k] - e3r[k]
        dd = dd + d * d
      dnv = _allsum(dd)
      wn = wlo[n] if n < _L else whi[n - _L]
      c = wn * jnp.sign(d3v - dnv)
      coef = coef + c
      for k in range(ND):
        accrow[k] = accrow[k] + c * crow[k]

    one_m = jnp.float32(1.0) - coef
    lacc = None
    for k in range(ND):
      outk = e2r[k] * one_m + accrow[k]
      d = outk - e3r[k]
      lacc = d * d if lacc is None else lacc + d * d
    sq = _allsum(lacc)

    lane = lax.iota(jnp.int32, _L)
    lv = jnp.where(lane == lax.rem(q, _L), sq, carry)

    @pl.when(lax.rem(q, _L) == _L - 1)
    def _():
      lossv[pl.ds((q // _L) * _L, _L)] = lv

    return lv

  lax.fori_loop(0, QPW, per_query, jnp.zeros((_L,), jnp.float32))
  pltpu.sync_copy(lossv, out_h.at[pl.ds(base, QPW)])


def kernel(e1s, e2s, e3s, r1s, r2s, node_embedding, node_type,
           node_neighbors, rel_neighbors, node_weight, rel_weight,
           type_weight, rel_eye):
  B = e2s.shape[0]
  NN = node_neighbors.shape[1]
  D = node_embedding.shape[1]
  QPW = B // _NW

  counts = jnp.sum(rel_eye[r1s], axis=0)
  edge_table = rel_weight + rel_weight * counts
  edge_pad = jnp.zeros((128,), jnp.float32).at[: edge_table.shape[0]].set(
      edge_table)
  typew_pad = jnp.zeros((128,), jnp.float32).at[: type_weight.shape[0]].set(
      type_weight)

  mesh = plsc.VectorSubcoreMesh(core_axis_name="c", subcore_axis_name="s")
  run = functools.partial(
      pl.kernel,
      mesh=mesh,
      compiler_params=pltpu.CompilerParams(use_tc_tiling_on_sc=False),
      out_type=jax.ShapeDtypeStruct((B,), jnp.float32),
      scratch_types=[
          pltpu.VMEM((QPW,), jnp.int32),
          pltpu.VMEM((QPW,), jnp.int32),
          pltpu.VMEM((QPW, NN), jnp.int32),
          pltpu.VMEM((QPW, NN), jnp.int32),
          pltpu.VMEM((QPW, D), jnp.float32),
          pltpu.VMEM((QPW, D), jnp.float32),
          pltpu.VMEM((NN, D), jnp.float32),
          pltpu.VMEM((NN,), jnp.float32),
          pltpu.VMEM((NN,), jnp.int32),
          pltpu.VMEM((NN,), jnp.int32),
          pltpu.VMEM((NN,), jnp.float32),
          pltpu.VMEM((NN,), jnp.float32),
          pltpu.VMEM((QPW,), jnp.float32),
          pltpu.SemaphoreType.DMA,
      ],
  )(_sc_body)

  sq = run(e2s.astype(jnp.int32), e3s.astype(jnp.int32), node_embedding,
           node_type.astype(jnp.int32), node_neighbors.astype(jnp.int32),
           rel_neighbors.astype(jnp.int32), node_weight, edge_pad, typew_pad)
  return jnp.mean(jnp.sqrt(sq + 1e-12))
